# baseline (speedup 1.0000x reference)
import math

import jax
import jax.numpy as jnp
import numpy as np
from jax.experimental import pallas as pl
from jax.experimental.pallas import tpu as pltpu

EPS = 1e-06
S = 0.025
ALPHA = 0.98
DELTA = 2.0

CHUNK = 256
BLOCK_T = 2048
LANES = 128


def _pcen(xb, m):
    return jnp.sqrt(
        xb * jnp.exp(-ALPHA * jnp.log(m + EPS)) + DELTA
    ) - np.float32(math.sqrt(DELTA))


def _pcen_kernel(x_ref, o_ref, l_scr, d_scr, p_scr, m_scr):
    k = pl.program_id(1)
    first = k == 0
    C = CHUNK

    @pl.when(first)
    def _init():
        ln_a = np.float32(math.log(1.0 - S))
        ii = jax.lax.broadcasted_iota(jnp.int32, (C, C), 0)
        jj = jax.lax.broadcasted_iota(jnp.int32, (C, C), 1)
        di = (ii - jj).astype(jnp.float32)
        l_scr[...] = jnp.where(di >= 0.0, S * jnp.exp(di * ln_a), 0.0)
        ir = jax.lax.broadcasted_iota(jnp.int32, (C, LANES), 0).astype(jnp.float32)
        d_scr[...] = (1.0 - S) * jnp.exp(ir * ln_a)
        p_scr[...] = jnp.exp((ir + 1.0) * ln_a)

    l_mat = l_scr[...]
    p_vec = p_scr[...]

    xb = x_ref[0, 0:C, :]
    vec = jnp.where(first, d_scr[...], p_vec)
    m_prev = jnp.where(first, xb[0:1, :], m_scr[...])
    m = jax.lax.dot_general(
        l_mat, xb, (((1,), (0,)), ((), ())),
        preferred_element_type=jnp.float32,
        precision=jax.lax.Precision.HIGHEST,
    ) + vec * m_prev
    o_ref[0, 0:C, :] = _pcen(xb, m)
    m_prev = m[C - 1:C, :]

    for c in range(1, BLOCK_T // C):
        xb = x_ref[0, c * C:(c + 1) * C, :]
        m = jax.lax.dot_general(
            l_mat, xb, (((1,), (0,)), ((), ())),
            preferred_element_type=jnp.float32,
            precision=jax.lax.Precision.HIGHEST,
        ) + p_vec * m_prev
        o_ref[0, c * C:(c + 1) * C, :] = _pcen(xb, m)
        m_prev = m[C - 1:C, :]

    m_scr[...] = m_prev


def _pcen_call(x):
    B, T, F = x.shape
    return pl.pallas_call(
        _pcen_kernel,
        grid=(B, T // BLOCK_T),
        in_specs=[pl.BlockSpec((1, BLOCK_T, F), lambda b, t: (b, t, 0))],
        out_specs=pl.BlockSpec((1, BLOCK_T, F), lambda b, t: (b, t, 0)),
        out_shape=jax.ShapeDtypeStruct((B, T, F), jnp.float32),
        scratch_shapes=[
            pltpu.VMEM((CHUNK, CHUNK), jnp.float32),
            pltpu.VMEM((CHUNK, F), jnp.float32),
            pltpu.VMEM((CHUNK, F), jnp.float32),
            pltpu.VMEM((1, F), jnp.float32),
        ],
        compiler_params=pltpu.CompilerParams(
            dimension_semantics=("arbitrary", "arbitrary"),
        ),
    )(x)


def kernel(x):
    devs = jax.devices()
    if len(devs) >= 2 and x.shape[0] % 2 == 0:
        mesh = jax.sharding.Mesh(np.array(devs[:2]), ("c",))
        xs = jax.device_put(
            x, jax.sharding.NamedSharding(mesh, jax.sharding.PartitionSpec("c"))
        )
        f = jax.shard_map(
            _pcen_call,
            mesh=mesh,
            in_specs=jax.sharding.PartitionSpec("c"),
            out_specs=jax.sharding.PartitionSpec("c"),
            check_vma=False,
        )
        return f(xs)
    return _pcen_call(x)


# speedup vs baseline: 4.1063x; 4.1063x over previous
import math

import jax
import jax.numpy as jnp
import numpy as np
from jax.experimental import pallas as pl
from jax.experimental.pallas import tpu as pltpu

EPS = 1e-06
S = 0.025
ALPHA = 0.98
DELTA = 2.0

CHUNK = 256
BLOCK_T = 2048
LANES = 128


def _pcen(xb, m):
    return jnp.sqrt(
        xb * jnp.exp(-ALPHA * jnp.log(m + EPS)) + DELTA
    ) - np.float32(math.sqrt(DELTA))


def _pcen_kernel(x_ref, o_ref, l_scr, d_scr, p_scr, m_scr):
    k = pl.program_id(1)
    first = k == 0
    C = CHUNK

    @pl.when(first)
    def _init():
        ln_a = np.float32(math.log(1.0 - S))
        ii = jax.lax.broadcasted_iota(jnp.int32, (C, C), 0)
        jj = jax.lax.broadcasted_iota(jnp.int32, (C, C), 1)
        di = (ii - jj).astype(jnp.float32)
        l_scr[...] = jnp.where(di >= 0.0, S * jnp.exp(di * ln_a), 0.0)
        ir = jax.lax.broadcasted_iota(jnp.int32, (C, LANES), 0).astype(jnp.float32)
        d_scr[...] = (1.0 - S) * jnp.exp(ir * ln_a)
        p_scr[...] = jnp.exp((ir + 1.0) * ln_a)

    l_mat = l_scr[...]
    p_vec = p_scr[...]

    xb = x_ref[0, 0:C, :]
    vec = jnp.where(first, d_scr[...], p_vec)
    m_prev = jnp.where(first, xb[0:1, :], m_scr[...])
    m = jax.lax.dot_general(
        l_mat, xb, (((1,), (0,)), ((), ())),
        preferred_element_type=jnp.float32,
    ) + vec * m_prev
    o_ref[0, 0:C, :] = _pcen(xb, m)
    m_prev = m[C - 1:C, :]

    for c in range(1, BLOCK_T // C):
        xb = x_ref[0, c * C:(c + 1) * C, :]
        m = jax.lax.dot_general(
            l_mat, xb, (((1,), (0,)), ((), ())),
            preferred_element_type=jnp.float32,
            ) + p_vec * m_prev
        o_ref[0, c * C:(c + 1) * C, :] = _pcen(xb, m)
        m_prev = m[C - 1:C, :]

    m_scr[...] = m_prev


def _pcen_call(x):
    B, T, F = x.shape
    return pl.pallas_call(
        _pcen_kernel,
        grid=(B, T // BLOCK_T),
        in_specs=[pl.BlockSpec((1, BLOCK_T, F), lambda b, t: (b, t, 0))],
        out_specs=pl.BlockSpec((1, BLOCK_T, F), lambda b, t: (b, t, 0)),
        out_shape=jax.ShapeDtypeStruct((B, T, F), jnp.float32),
        scratch_shapes=[
            pltpu.VMEM((CHUNK, CHUNK), jnp.float32),
            pltpu.VMEM((CHUNK, F), jnp.float32),
            pltpu.VMEM((CHUNK, F), jnp.float32),
            pltpu.VMEM((1, F), jnp.float32),
        ],
        compiler_params=pltpu.CompilerParams(
            dimension_semantics=("arbitrary", "arbitrary"),
        ),
    )(x)


def kernel(x):
    return _pcen_call(x)


# speedup vs baseline: 4.4703x; 1.0886x over previous
import math

import jax
import jax.numpy as jnp
import numpy as np
from jax.experimental import pallas as pl
from jax.experimental.pallas import tpu as pltpu

EPS = 1e-06
S = 0.025
ALPHA = 0.98
DELTA = 2.0

CHUNK = 256
BLOCK_T = 2048
LANES = 128


def _pcen(xb, m):
    w = jax.lax.exp2(jnp.log2(m + EPS) * np.float32(-ALPHA))
    y = xb * w + DELTA
    return y * jax.lax.rsqrt(y) - np.float32(math.sqrt(DELTA))


def _pcen_kernel(x_ref, o_ref, l_scr, d_scr, p_scr, m_scr):
    k = pl.program_id(1)
    first = k == 0
    C = CHUNK

    @pl.when(first & (pl.program_id(0) == 0))
    def _init():
        ln_a = np.float32(math.log(1.0 - S))
        ii = jax.lax.broadcasted_iota(jnp.int32, (C, C), 0)
        jj = jax.lax.broadcasted_iota(jnp.int32, (C, C), 1)
        di = (ii - jj).astype(jnp.float32)
        l_scr[...] = jnp.where(di >= 0.0, S * jnp.exp(di * ln_a), 0.0)
        ir = jax.lax.broadcasted_iota(jnp.int32, (C, LANES), 0).astype(jnp.float32)
        d_scr[...] = (1.0 - S) * jnp.exp(ir * ln_a)
        p_scr[...] = jnp.exp((ir + 1.0) * ln_a)

    l_mat = l_scr[...]
    p_vec = p_scr[...]

    xb = x_ref[0, 0:C, :]
    vec = jnp.where(first, d_scr[...], p_vec)
    m_prev = jnp.where(first, xb[0:1, :], m_scr[...])
    m = jax.lax.dot_general(
        l_mat, xb, (((1,), (0,)), ((), ())),
        preferred_element_type=jnp.float32,
    ) + vec * m_prev
    o_ref[0, 0:C, :] = _pcen(xb, m)
    m_prev = m[C - 1:C, :]

    for c in range(1, BLOCK_T // C):
        xb = x_ref[0, c * C:(c + 1) * C, :]
        m = jax.lax.dot_general(
            l_mat, xb, (((1,), (0,)), ((), ())),
            preferred_element_type=jnp.float32,
            ) + p_vec * m_prev
        o_ref[0, c * C:(c + 1) * C, :] = _pcen(xb, m)
        m_prev = m[C - 1:C, :]

    m_scr[...] = m_prev


def _pcen_call(x):
    B, T, F = x.shape
    return pl.pallas_call(
        _pcen_kernel,
        grid=(B, T // BLOCK_T),
        in_specs=[pl.BlockSpec((1, BLOCK_T, F), lambda b, t: (b, t, 0))],
        out_specs=pl.BlockSpec((1, BLOCK_T, F), lambda b, t: (b, t, 0)),
        out_shape=jax.ShapeDtypeStruct((B, T, F), jnp.float32),
        scratch_shapes=[
            pltpu.VMEM((CHUNK, CHUNK), jnp.float32),
            pltpu.VMEM((CHUNK, F), jnp.float32),
            pltpu.VMEM((CHUNK, F), jnp.float32),
            pltpu.VMEM((1, F), jnp.float32),
        ],
        compiler_params=pltpu.CompilerParams(
            dimension_semantics=("arbitrary", "arbitrary"),
        ),
    )(x)


def kernel(x):
    return _pcen_call(x)


# speedup vs baseline: 5.8623x; 1.3114x over previous
import math

import jax
import jax.numpy as jnp
import numpy as np
from jax.experimental import pallas as pl
from jax.experimental.pallas import tpu as pltpu

EPS = 1e-06
S = 0.025
ALPHA = 0.98
DELTA = 2.0

CHUNK = 256
BLOCK_T = 4096
LANES = 128


def _pcen(xb, m):
    w = jax.lax.exp2(jnp.log2(m + EPS) * np.float32(-ALPHA))
    y = xb * w + DELTA
    return y * jax.lax.rsqrt(y) - np.float32(math.sqrt(DELTA))


def _pcen_kernel(x_ref, o_ref, l_scr, d_scr, p_scr, m_scr):
    k = pl.program_id(1)
    first = k == 0
    C = CHUNK

    @pl.when(first & (pl.program_id(0) == 0))
    def _init():
        ln_a = np.float32(math.log(1.0 - S))
        ii = jax.lax.broadcasted_iota(jnp.int32, (C, C), 0)
        jj = jax.lax.broadcasted_iota(jnp.int32, (C, C), 1)
        di = (ii - jj).astype(jnp.float32)
        l_scr[...] = jnp.where(di >= 0.0, S * jnp.exp(di * ln_a), 0.0)
        ir = jax.lax.broadcasted_iota(jnp.int32, (C, LANES), 0).astype(jnp.float32)
        d_scr[...] = (1.0 - S) * jnp.exp(ir * ln_a)
        p_scr[...] = jnp.exp((ir + 1.0) * ln_a)

    l_mat = l_scr[...]
    p_vec = p_scr[...]

    xb = x_ref[0, 0:C, :]
    vec = jnp.where(first, d_scr[...], p_vec)
    m_prev = jnp.where(first, xb[0:1, :], m_scr[...])
    m = jax.lax.dot_general(
        l_mat, xb, (((1,), (0,)), ((), ())),
        preferred_element_type=jnp.float32,
    ) + vec * m_prev
    o_ref[0, 0:C, :] = _pcen(xb, m)
    m_prev = m[C - 1:C, :]

    for c in range(1, BLOCK_T // C):
        xb = x_ref[0, c * C:(c + 1) * C, :]
        m = jax.lax.dot_general(
            l_mat, xb, (((1,), (0,)), ((), ())),
            preferred_element_type=jnp.float32,
            ) + p_vec * m_prev
        o_ref[0, c * C:(c + 1) * C, :] = _pcen(xb, m)
        m_prev = m[C - 1:C, :]

    m_scr[...] = m_prev


def _pcen_call(x):
    B, T, F = x.shape
    return pl.pallas_call(
        _pcen_kernel,
        grid=(B, T // BLOCK_T),
        in_specs=[pl.BlockSpec((1, BLOCK_T, F), lambda b, t: (b, t, 0))],
        out_specs=pl.BlockSpec((1, BLOCK_T, F), lambda b, t: (b, t, 0)),
        out_shape=jax.ShapeDtypeStruct((B, T, F), jnp.float32),
        scratch_shapes=[
            pltpu.VMEM((CHUNK, CHUNK), jnp.float32),
            pltpu.VMEM((CHUNK, F), jnp.float32),
            pltpu.VMEM((CHUNK, F), jnp.float32),
            pltpu.VMEM((1, F), jnp.float32),
        ],
        compiler_params=pltpu.CompilerParams(
            dimension_semantics=("arbitrary", "arbitrary"),
        ),
    )(x)


def kernel(x):
    return _pcen_call(x)


# speedup vs baseline: 6.9445x; 1.1846x over previous
import math

import jax
import jax.numpy as jnp
import numpy as np
from jax.experimental import pallas as pl
from jax.experimental.pallas import tpu as pltpu

EPS = 1e-06
S = 0.025
ALPHA = 0.98
DELTA = 2.0

CHUNK = 256
BLOCK_T = 8192
LANES = 128


def _pcen(xb, m):
    w = jax.lax.exp2(jnp.log2(m + EPS) * np.float32(-ALPHA))
    y = xb * w + DELTA
    return y * jax.lax.rsqrt(y) - np.float32(math.sqrt(DELTA))


def _pcen_kernel(x_ref, o_ref, l_scr, d_scr, p_scr, m_scr):
    k = pl.program_id(1)
    first = k == 0
    C = CHUNK

    @pl.when(first & (pl.program_id(0) == 0))
    def _init():
        ln_a = np.float32(math.log(1.0 - S))
        ii = jax.lax.broadcasted_iota(jnp.int32, (C, C), 0)
        jj = jax.lax.broadcasted_iota(jnp.int32, (C, C), 1)
        di = (ii - jj).astype(jnp.float32)
        l_scr[...] = jnp.where(di >= 0.0, S * jnp.exp(di * ln_a), 0.0)
        ir = jax.lax.broadcasted_iota(jnp.int32, (C, LANES), 0).astype(jnp.float32)
        d_scr[...] = (1.0 - S) * jnp.exp(ir * ln_a)
        p_scr[...] = jnp.exp((ir + 1.0) * ln_a)

    l_mat = l_scr[...]
    p_vec = p_scr[...]

    xb = x_ref[0, 0:C, :]
    vec = jnp.where(first, d_scr[...], p_vec)
    m_prev = jnp.where(first, xb[0:1, :], m_scr[...])
    m = jax.lax.dot_general(
        l_mat, xb, (((1,), (0,)), ((), ())),
        preferred_element_type=jnp.float32,
    ) + vec * m_prev
    o_ref[0, 0:C, :] = _pcen(xb, m)
    m_prev = m[C - 1:C, :]

    for c in range(1, BLOCK_T // C):
        xb = x_ref[0, c * C:(c + 1) * C, :]
        m = jax.lax.dot_general(
            l_mat, xb, (((1,), (0,)), ((), ())),
            preferred_element_type=jnp.float32,
            ) + p_vec * m_prev
        o_ref[0, c * C:(c + 1) * C, :] = _pcen(xb, m)
        m_prev = m[C - 1:C, :]

    m_scr[...] = m_prev


def _pcen_call(x):
    B, T, F = x.shape
    return pl.pallas_call(
        _pcen_kernel,
        grid=(B, T // BLOCK_T),
        in_specs=[pl.BlockSpec((1, BLOCK_T, F), lambda b, t: (b, t, 0))],
        out_specs=pl.BlockSpec((1, BLOCK_T, F), lambda b, t: (b, t, 0)),
        out_shape=jax.ShapeDtypeStruct((B, T, F), jnp.float32),
        scratch_shapes=[
            pltpu.VMEM((CHUNK, CHUNK), jnp.float32),
            pltpu.VMEM((CHUNK, F), jnp.float32),
            pltpu.VMEM((CHUNK, F), jnp.float32),
            pltpu.VMEM((1, F), jnp.float32),
        ],
        compiler_params=pltpu.CompilerParams(
            dimension_semantics=("arbitrary", "arbitrary"),
        ),
    )(x)


def kernel(x):
    return _pcen_call(x)
